# Initial kernel scaffold; baseline (speedup 1.0000x reference)
import jax
import jax.numpy as jnp
from jax.experimental import pallas as pl


def kernel(memory, node_idx, raw_messages, W1, b1, W2, b2, W_ih, b_ih, W_hh, b_hh):
    raise NotImplementedError("write your pallas kernel here")


# speedup vs baseline: 3.1507x; 3.1507x over previous
import functools

import jax
import jax.numpy as jnp
from jax import lax
from jax.experimental import pallas as pl
from jax.experimental.pallas import tpu as pltpu
from jax.experimental.pallas import tpu_sc as plsc

_N = 100000
_D = 500
_MD = 100
_B = 16384

_NC = 2
_NS = 16
_NW = _NC * _NS
_BPW = _B // _NW
_CH = 64
_NCH = _BPW // _CH


_PIECES = ((0, 0), (128, 128), (256, 256), (372, 384))


def _sc_gather_body(mem_hbm, tail_hbm, rm_hbm, idx_hbm, w_hbm, h_out, rm2_out,
                    idx_v, w_v, hbuf, rmbuf, sem, sem2):
    wid = lax.axis_index("s") * _NC + lax.axis_index("c")
    base = wid * _BPW
    pltpu.sync_copy(idx_hbm.at[pl.ds(base, _BPW)], idx_v)
    pltpu.sync_copy(w_hbm.at[pl.ds(base, _BPW)], w_v)
    for c in range(_NCH):
        ids = idx_v.at[pl.ds(c * _CH, _CH)]
        cps = [pltpu.async_copy(
                   mem_hbm.at[ids, pl.ds(src, 128)], hbuf.at[k], sem)
               for k, (src, _) in enumerate(_PIECES[:3])]
        cps.append(pltpu.async_copy(tail_hbm.at[ids], hbuf.at[3], sem))
        cp_m = pltpu.async_copy(
            rm_hbm.at[w_v.at[pl.ds(c * _CH, _CH)]], rmbuf, sem2)
        rows = pl.ds(base + c * _CH, _CH)
        for k, (_, dst) in enumerate(_PIECES):
            cps[k].wait()
            pltpu.sync_copy(hbuf.at[k], h_out.at[rows, pl.ds(dst, 128)])
        cp_m.wait()
        pltpu.sync_copy(rmbuf, rm2_out.at[rows])


@functools.cache
def _sc_gather():
    return pl.kernel(
        _sc_gather_body,
        out_type=[jax.ShapeDtypeStruct((_B, 512), jnp.float32),
                  jax.ShapeDtypeStruct((_B, 128), jnp.float32)],
        mesh=plsc.VectorSubcoreMesh(core_axis_name="c", subcore_axis_name="s",
                                    num_cores=_NC, num_subcores=_NS),
        scratch_types=[
            pltpu.VMEM((_BPW,), jnp.int32),
            pltpu.VMEM((_BPW,), jnp.int32),
            pltpu.VMEM((4, _CH, 128), jnp.float32),
            pltpu.VMEM((_CH, 128), jnp.float32),
            pltpu.SemaphoreType.DMA,
            pltpu.SemaphoreType.DMA,
        ],
    )


_BM = 256


def _tc_body(h_ref, rm_ref, w1_ref, b1_ref, w2_ref, b2_ref,
             wir_ref, wiz_ref, win_ref, bi_ref,
             whr_ref, whz_ref, whn_ref, bh_ref, out_ref):
    f32 = jnp.float32
    cdims = (((1,), (1,)), ((), ()))
    hp = h_ref[...]
    h = jnp.concatenate([hp[:, :384], hp[:, 396:512]], axis=1)
    m = jax.nn.relu(
        lax.dot_general(rm_ref[:, :_MD], w1_ref[...], cdims,
                        preferred_element_type=f32) + b1_ref[...])
    m = lax.dot_general(m, w2_ref[...], cdims,
                        preferred_element_type=f32) + b2_ref[...]
    gir = lax.dot_general(m, wir_ref[...], cdims,
                          preferred_element_type=f32) + bi_ref[0:1, :]
    giz = lax.dot_general(m, wiz_ref[...], cdims,
                          preferred_element_type=f32) + bi_ref[1:2, :]
    gin = lax.dot_general(m, win_ref[...], cdims,
                          preferred_element_type=f32) + bi_ref[2:3, :]
    ghr = lax.dot_general(h, whr_ref[...], cdims,
                          preferred_element_type=f32) + bh_ref[0:1, :]
    ghz = lax.dot_general(h, whz_ref[...], cdims,
                          preferred_element_type=f32) + bh_ref[1:2, :]
    ghn = lax.dot_general(h, whn_ref[...], cdims,
                          preferred_element_type=f32) + bh_ref[2:3, :]
    r = jax.nn.sigmoid(gir + ghr)
    z = jax.nn.sigmoid(giz + ghz)
    n = jnp.tanh(gin + r * ghn)
    out_ref[...] = (1.0 - z) * n + z * h


def _tc_call(h, rm2, W1, b1, W2, b2, W_ih, b_ih, W_hh, b_hh):
    wir, wiz, win = W_ih[:_D], W_ih[_D:2 * _D], W_ih[2 * _D:]
    whr, whz, whn = W_hh[:_D], W_hh[_D:2 * _D], W_hh[2 * _D:]
    bi = b_ih.reshape(3, _D)
    bh = b_hh.reshape(3, _D)
    full = lambda s: pl.BlockSpec(s, lambda i: (0, 0))
    return pl.pallas_call(
        _tc_body,
        grid=(_B // _BM,),
        in_specs=[
            pl.BlockSpec((_BM, 512), lambda i: (i, 0)),
            pl.BlockSpec((_BM, 128), lambda i: (i, 0)),
            full((_MD // 2, _MD)), full((1, _MD // 2)),
            full((_MD, _MD // 2)), full((1, _MD)),
            full((_D, _MD)), full((_D, _MD)), full((_D, _MD)), full((3, _D)),
            full((_D, _D)), full((_D, _D)), full((_D, _D)), full((3, _D)),
        ],
        out_specs=pl.BlockSpec((_BM, _D), lambda i: (i, 0)),
        out_shape=jax.ShapeDtypeStruct((_B, _D), jnp.float32),
    )(h, rm2, W1, b1.reshape(1, -1), W2, b2.reshape(1, -1),
      wir, wiz, win, bi, whr, whz, whn, bh)


def kernel(memory, node_idx, raw_messages, W1, b1, W2, b2,
           W_ih, b_ih, W_hh, b_hh):
    idx = node_idx.astype(jnp.int32)
    w = jnp.zeros((_N,), jnp.int32).at[idx].set(
        jnp.arange(_B, dtype=jnp.int32))[idx]
    rm_p = jnp.pad(raw_messages, ((0, 0), (0, 128 - _MD)))
    tail = lax.slice(memory, (0, 372), (_N, _D))
    h, rm2 = _sc_gather()(memory, tail, rm_p, idx, w)
    return _tc_call(h, rm2, W1, b1, W2, b2, W_ih, b_ih, W_hh, b_hh)
